# Initial kernel scaffold; baseline (speedup 1.0000x reference)
import jax
import jax.numpy as jnp
from jax.experimental import pallas as pl


def kernel(x, edge_index, edge_type, emb, W1, b1, W2, b2):
    raise NotImplementedError("write your pallas kernel here")


# speedup vs baseline: 1.6897x; 1.6897x over previous
import functools

import jax
import jax.numpy as jnp
from jax import lax
from jax.experimental import pallas as pl
from jax.experimental.pallas import tpu as pltpu
from jax.experimental.pallas import tpu_sc as plsc

N = 10000
E = 320000
D = 128
ET = 16

NC = 2
NS = 16
NW = NC * NS
L = 16

CHUNK = 128
PER_W = 10240
EP = PER_W * NW
NCHUNK = PER_W // CHUNK


def _ab_body(x_ref, wa_ref, wb_ref, a_ref, b_ref):
    xb = x_ref[...]
    a_ref[...] = jnp.dot(xb, wa_ref[...], preferred_element_type=jnp.float32)
    b_ref[...] = jnp.dot(xb, wb_ref[...], preferred_element_type=jnp.float32)


def _c_body(emb_ref, wc_ref, b1_ref, c_ref):
    c_ref[...] = (
        jnp.dot(emb_ref[...], wc_ref[...], preferred_element_type=jnp.float32)
        + b1_ref[...]
    )


def _precompute_tables(x, emb, W1, b1):
    w1a = W1[0:D, :]
    w1b = W1[D:2 * D, :]
    w1c = W1[2 * D:, :]
    blk = 1000
    a, b = pl.pallas_call(
        _ab_body,
        grid=(N // blk,),
        in_specs=[
            pl.BlockSpec((blk, D), lambda i: (i, 0)),
            pl.BlockSpec((D, D), lambda i: (0, 0)),
            pl.BlockSpec((D, D), lambda i: (0, 0)),
        ],
        out_specs=[
            pl.BlockSpec((blk, D), lambda i: (i, 0)),
            pl.BlockSpec((blk, D), lambda i: (i, 0)),
        ],
        out_shape=[
            jax.ShapeDtypeStruct((N, D), jnp.float32),
            jax.ShapeDtypeStruct((N, D), jnp.float32),
        ],
    )(x, w1a, w1b)
    c = pl.pallas_call(
        _c_body,
        out_shape=jax.ShapeDtypeStruct((512, D), jnp.float32),
    )(emb, w1c, b1.reshape(1, D))
    return a, b, c


def _sc_edge_kernel(a_hbm, b_hbm, c_hbm, src_hbm, dst_hbm, et_hbm,
                    w2_hbm, b2_hbm, logits_hbm, scores_hbm,
                    sidx, didx, tidx, buf_a, buf_b, buf_c,
                    lbuf, sbuf, w2_v, b2_v, sem):
    wid = lax.axis_index("s") * NC + lax.axis_index("c")
    w_base = wid * PER_W

    pltpu.sync_copy(w2_hbm, w2_v)
    pltpu.sync_copy(b2_hbm, b2_v)
    w2c = [w2_v[pl.ds(k * L, L)] for k in range(D // L)]
    b2s = b2_v[...][0]
    lane = lax.broadcasted_iota(jnp.int32, (L,), 0)

    def chunk_body(ci, carry):
        base = w_base + ci * CHUNK
        pltpu.sync_copy(src_hbm.at[pl.ds(base, CHUNK)], sidx)
        pltpu.sync_copy(dst_hbm.at[pl.ds(base, CHUNK)], didx)
        pltpu.sync_copy(et_hbm.at[pl.ds(base, CHUNK)], tidx)
        ca = pltpu.async_copy(a_hbm.at[sidx], buf_a, sem)
        cb = pltpu.async_copy(b_hbm.at[didx], buf_b, sem)
        cc = pltpu.async_copy(c_hbm.at[tidx], buf_c, sem)
        ca.wait()
        cb.wait()
        cc.wait()

        def group_body(g, carry2):
            logv = jnp.zeros((L,), jnp.float32)
            for j in range(L):
                e = g * L + j
                acc = jnp.zeros((L,), jnp.float32)
                for k in range(D // L):
                    va = buf_a[e, pl.ds(k * L, L)]
                    vb = buf_b[e, pl.ds(k * L, L)]
                    vc = buf_c[e, pl.ds(k * L, L)]
                    acc = acc + jnp.maximum(va + vb + vc, 0.0) * w2c[k]
                logit = jnp.sum(acc) + b2s
                logv = jnp.where(lane == j, logit, logv)
            lbuf[pl.ds(g * L, L)] = logv
            sbuf[pl.ds(g * L, L)] = 1.0 / (1.0 + jnp.exp(-logv))
            return carry2

        lax.fori_loop(0, CHUNK // L, group_body, 0)
        pltpu.sync_copy(lbuf, logits_hbm.at[pl.ds(base, CHUNK)])
        pltpu.sync_copy(sbuf, scores_hbm.at[pl.ds(base, CHUNK)])
        return carry

    lax.fori_loop(0, NCHUNK, chunk_body, 0)


@functools.partial(
    pl.kernel,
    out_type=[
        jax.ShapeDtypeStruct((EP,), jnp.float32),
        jax.ShapeDtypeStruct((EP,), jnp.float32),
    ],
    mesh=plsc.VectorSubcoreMesh(core_axis_name="c", subcore_axis_name="s"),
    compiler_params=pltpu.CompilerParams(needs_layout_passes=False),
    scratch_types=[
        pltpu.VMEM((CHUNK,), jnp.int32),
        pltpu.VMEM((CHUNK,), jnp.int32),
        pltpu.VMEM((CHUNK,), jnp.int32),
        pltpu.VMEM((CHUNK, D), jnp.float32),
        pltpu.VMEM((CHUNK, D), jnp.float32),
        pltpu.VMEM((CHUNK, D), jnp.float32),
        pltpu.VMEM((CHUNK,), jnp.float32),
        pltpu.VMEM((CHUNK,), jnp.float32),
        pltpu.VMEM((D,), jnp.float32),
        pltpu.VMEM((L,), jnp.float32),
        pltpu.SemaphoreType.DMA,
    ],
)
def _sc_edges(a, b, c, src, dst, et, w2, b2, logits, scores, *scratch):
    _sc_edge_kernel(a, b, c, src, dst, et, w2, b2, logits, scores, *scratch)


def kernel(x, edge_index, edge_type, emb, W1, b1, W2, b2):
    a, b, c = _precompute_tables(x, emb, W1, b1)

    pad = EP - E
    src = jnp.concatenate([edge_index[0], jnp.zeros((pad,), jnp.int32)])
    dst = jnp.concatenate([edge_index[1], jnp.zeros((pad,), jnp.int32)])
    et = jnp.concatenate([edge_type, jnp.zeros((pad,), jnp.int32)])

    w2 = W2.reshape(D)
    b2v = jnp.broadcast_to(b2.reshape(1), (L,)).astype(jnp.float32)

    logits_p, scores_p = _sc_edges(a, b, c, src, dst, et, w2, b2v)
    return logits_p[:E], scores_p[:E]


# speedup vs baseline: 3.0489x; 1.8045x over previous
import functools

import jax
import jax.numpy as jnp
from jax import lax
from jax.experimental import pallas as pl
from jax.experimental.pallas import tpu as pltpu
from jax.experimental.pallas import tpu_sc as plsc

N = 10000
E = 320000
D = 128
ET = 16

NC = 2
NS = 16
NW = NC * NS
L = 16

CHUNK = 80
PER_W = 10240
EP = PER_W * NW
NCHUNK = PER_W // CHUNK


def _ab_body(x_ref, wa_ref, wb_ref, a_ref, b_ref):
    xb = x_ref[...]
    a_ref[...] = jnp.dot(xb, wa_ref[...], preferred_element_type=jnp.float32)
    b_ref[...] = jnp.dot(xb, wb_ref[...], preferred_element_type=jnp.float32)


def _c_body(emb_ref, wc_ref, b1_ref, c_ref):
    c_ref[...] = (
        jnp.dot(emb_ref[...], wc_ref[...], preferred_element_type=jnp.float32)
        + b1_ref[...]
    )


def _precompute_tables(x, emb, W1, b1):
    w1a = W1[0:D, :]
    w1b = W1[D:2 * D, :]
    w1c = W1[2 * D:, :]
    blk = 1000
    a, b = pl.pallas_call(
        _ab_body,
        grid=(N // blk,),
        in_specs=[
            pl.BlockSpec((blk, D), lambda i: (i, 0)),
            pl.BlockSpec((D, D), lambda i: (0, 0)),
            pl.BlockSpec((D, D), lambda i: (0, 0)),
        ],
        out_specs=[
            pl.BlockSpec((blk, D), lambda i: (i, 0)),
            pl.BlockSpec((blk, D), lambda i: (i, 0)),
        ],
        out_shape=[
            jax.ShapeDtypeStruct((N, D), jnp.float32),
            jax.ShapeDtypeStruct((N, D), jnp.float32),
        ],
    )(x, w1a, w1b)
    c = pl.pallas_call(
        _c_body,
        out_shape=jax.ShapeDtypeStruct((512, D), jnp.float32),
    )(emb, w1c, b1.reshape(1, D))
    return a, b, c


def _sc_edge_kernel(a_hbm, b_hbm, c_hbm, src_hbm, dst_hbm, et_hbm,
                    w2_hbm, b2_hbm, logits_hbm, scores_hbm,
                    sidx, didx, tidx, buf_a, buf_b, buf_c,
                    lbuf, sbuf, w2_v, b2_v, gsem):
    wid = lax.axis_index("s") * NC + lax.axis_index("c")
    w_base = wid * PER_W

    pltpu.sync_copy(w2_hbm, w2_v)
    pltpu.sync_copy(b2_hbm, b2_v)
    pltpu.sync_copy(src_hbm.at[pl.ds(w_base, PER_W)], sidx)
    pltpu.sync_copy(dst_hbm.at[pl.ds(w_base, PER_W)], didx)
    pltpu.sync_copy(et_hbm.at[pl.ds(w_base, PER_W)], tidx)

    w2c = [w2_v[pl.ds(k * L, L)] for k in range(D // L)]
    b2s = b2_v[...][0]
    lane = lax.broadcasted_iota(jnp.int32, (L,), 0)

    def issue(ci, p):
        off = ci * CHUNK
        dsl = pl.ds(p * CHUNK, CHUNK)
        pltpu.async_copy(a_hbm.at[sidx.at[pl.ds(off, CHUNK)]],
                         buf_a.at[dsl], gsem.at[p])
        pltpu.async_copy(b_hbm.at[didx.at[pl.ds(off, CHUNK)]],
                         buf_b.at[dsl], gsem.at[p])
        pltpu.async_copy(c_hbm.at[tidx.at[pl.ds(off, CHUNK)]],
                         buf_c.at[dsl], gsem.at[p])

    def drain(p):
        dsl = pl.ds(p * CHUNK, CHUNK)
        dummy = a_hbm.at[pl.ds(0, CHUNK)]
        pltpu.make_async_copy(dummy, buf_a.at[dsl], gsem.at[p]).wait()
        pltpu.make_async_copy(dummy, buf_b.at[dsl], gsem.at[p]).wait()
        pltpu.make_async_copy(dummy, buf_c.at[dsl], gsem.at[p]).wait()

    issue(0, 0)

    def chunk_body(ci, carry):
        p = lax.rem(ci, 2)

        @pl.when(ci + 1 < NCHUNK)
        def _():
            issue(ci + 1, 1 - p)

        drain(p)
        ebase = p * CHUNK

        def group_body(g, carry2):
            logv = jnp.zeros((L,), jnp.float32)
            for j in range(L):
                e = ebase + g * L + j
                acc = jnp.zeros((L,), jnp.float32)
                for k in range(D // L):
                    va = buf_a[e, pl.ds(k * L, L)]
                    vb = buf_b[e, pl.ds(k * L, L)]
                    vc = buf_c[e, pl.ds(k * L, L)]
                    acc = acc + jnp.maximum(va + vb + vc, 0.0) * w2c[k]
                logit = jnp.sum(acc) + b2s
                logv = jnp.where(lane == j, logit, logv)
            obase = ci * CHUNK + g * L
            lbuf[pl.ds(obase, L)] = logv
            sbuf[pl.ds(obase, L)] = 1.0 / (1.0 + jnp.exp(-logv))
            return carry2

        lax.fori_loop(0, CHUNK // L, group_body, 0)
        return carry

    lax.fori_loop(0, NCHUNK, chunk_body, 0)
    pltpu.sync_copy(lbuf, logits_hbm.at[pl.ds(w_base, PER_W)])
    pltpu.sync_copy(sbuf, scores_hbm.at[pl.ds(w_base, PER_W)])


@functools.partial(
    pl.kernel,
    out_type=[
        jax.ShapeDtypeStruct((EP,), jnp.float32),
        jax.ShapeDtypeStruct((EP,), jnp.float32),
    ],
    mesh=plsc.VectorSubcoreMesh(core_axis_name="c", subcore_axis_name="s"),
    compiler_params=pltpu.CompilerParams(needs_layout_passes=False),
    scratch_types=[
        pltpu.VMEM((PER_W,), jnp.int32),
        pltpu.VMEM((PER_W,), jnp.int32),
        pltpu.VMEM((PER_W,), jnp.int32),
        pltpu.VMEM((2 * CHUNK, D), jnp.float32),
        pltpu.VMEM((2 * CHUNK, D), jnp.float32),
        pltpu.VMEM((2 * CHUNK, D), jnp.float32),
        pltpu.VMEM((PER_W,), jnp.float32),
        pltpu.VMEM((PER_W,), jnp.float32),
        pltpu.VMEM((D,), jnp.float32),
        pltpu.VMEM((L,), jnp.float32),
        pltpu.SemaphoreType.DMA((2,)),
    ],
)
def _sc_edges(a, b, c, src, dst, et, w2, b2, logits, scores, *scratch):
    _sc_edge_kernel(a, b, c, src, dst, et, w2, b2, logits, scores, *scratch)


def kernel(x, edge_index, edge_type, emb, W1, b1, W2, b2):
    a, b, c = _precompute_tables(x, emb, W1, b1)

    pad = EP - E
    src = jnp.concatenate([edge_index[0], jnp.zeros((pad,), jnp.int32)])
    dst = jnp.concatenate([edge_index[1], jnp.zeros((pad,), jnp.int32)])
    et = jnp.concatenate([edge_type, jnp.zeros((pad,), jnp.int32)])

    w2 = W2.reshape(D)
    b2v = jnp.broadcast_to(b2.reshape(1), (L,)).astype(jnp.float32)

    logits_p, scores_p = _sc_edges(a, b, c, src, dst, et, w2, b2v)
    return logits_p[:E], scores_p[:E]


# speedup vs baseline: 3.2720x; 1.0732x over previous
import functools

import jax
import jax.numpy as jnp
from jax import lax
from jax.experimental import pallas as pl
from jax.experimental.pallas import tpu as pltpu
from jax.experimental.pallas import tpu_sc as plsc

N = 10000
E = 320000
D = 128
ET = 16

NC = 2
NS = 16
NW = NC * NS
L = 16

CHUNK = 64
PER_W = 10240
EP = PER_W * NW
NCHUNK = PER_W // CHUNK
NVEC = PER_W // L
KB = 32


def _ab_body(x_ref, wa_ref, wb_ref, a_ref, b_ref):
    xb = x_ref[...]
    a_ref[...] = jnp.dot(xb, wa_ref[...], preferred_element_type=jnp.float32)
    b_ref[...] = jnp.dot(xb, wb_ref[...], preferred_element_type=jnp.float32)


def _c_body(emb_ref, wc_ref, b1_ref, c_ref):
    c_ref[...] = (
        jnp.dot(emb_ref[...], wc_ref[...], preferred_element_type=jnp.float32)
        + b1_ref[...]
    )


def _precompute_tables(x, emb, W1, b1):
    w1a = W1[0:D, :]
    w1b = W1[D:2 * D, :]
    w1c = W1[2 * D:, :]
    blk = 1000
    a, b = pl.pallas_call(
        _ab_body,
        grid=(N // blk,),
        in_specs=[
            pl.BlockSpec((blk, D), lambda i: (i, 0)),
            pl.BlockSpec((D, D), lambda i: (0, 0)),
            pl.BlockSpec((D, D), lambda i: (0, 0)),
        ],
        out_specs=[
            pl.BlockSpec((blk, D), lambda i: (i, 0)),
            pl.BlockSpec((blk, D), lambda i: (i, 0)),
        ],
        out_shape=[
            jax.ShapeDtypeStruct((N, D), jnp.float32),
            jax.ShapeDtypeStruct((N, D), jnp.float32),
        ],
    )(x, w1a, w1b)
    c = pl.pallas_call(
        _c_body,
        out_shape=jax.ShapeDtypeStruct((512, D), jnp.float32),
    )(emb, w1c, b1.reshape(1, D))
    return a, b, c


def _sc_edge_kernel(a_hbm, b_hbm, c_hbm, src_hbm, dst_hbm, et_hbm,
                    w2_hbm, b2_hbm, logits_hbm, scores_hbm,
                    sidx, didx, sidxp, didxp, etp, permq, lstage,
                    counts, cursors, buf_a, buf_b, buf_c,
                    etring, outring, w2_v, b2_v, w2r, gsem, osem):
    wid = lax.axis_index("s") * NC + lax.axis_index("c")
    w_base = wid * PER_W

    pltpu.sync_copy(w2_hbm, w2_v)
    pltpu.sync_copy(b2_hbm, b2_v)
    pltpu.sync_copy(src_hbm.at[pl.ds(w_base, PER_W)], sidx)
    pltpu.sync_copy(dst_hbm.at[pl.ds(w_base, PER_W)], didx)

    b2s = b2_v[...][0]
    lane = lax.broadcasted_iota(jnp.int32, (L,), 0)
    ones = jnp.ones((L,), jnp.int32)

    def zero_body(k, carry):
        counts[pl.ds(k * L, L)] = jnp.zeros((L,), jnp.int32)
        return carry

    lax.fori_loop(0, KB * L // L, zero_body, 0)

    def hist_body(t, carry):
        src_v = sidx[pl.ds(t * L, L)]
        cidx = jnp.right_shift(src_v, 9) * L + lane
        plsc.addupdate_scatter(counts, [cidx], ones)
        return carry

    lax.fori_loop(0, NVEC, hist_body, 0)

    def base_body(k, bstart):
        cv = counts[pl.ds(k * L, L)]
        cum = plsc.cumsum(cv)
        cursors[pl.ds(k * L, L)] = (cum - cv) + bstart
        return bstart + cum[L - 1]

    lax.fori_loop(0, 20, base_body, jnp.int32(0))

    pltpu.async_copy(et_hbm.at[pl.ds(w_base, CHUNK)],
                     etring.at[pl.ds(0, CHUNK)], osem.at[0])

    def scat_chunk(cc, carry):
        p = lax.rem(cc, 2)

        @pl.when(cc + 1 < NCHUNK)
        def _():
            pltpu.async_copy(
                et_hbm.at[pl.ds(w_base + (cc + 1) * CHUNK, CHUNK)],
                etring.at[pl.ds((1 - p) * CHUNK, CHUNK)], osem.at[1 - p])

        pltpu.make_async_copy(et_hbm.at[pl.ds(0, CHUNK)],
                              etring.at[pl.ds(p * CHUNK, CHUNK)],
                              osem.at[p]).wait()

        def scat_vec(g, carry2):
            t = cc * (CHUNK // L) + g
            src_v = sidx[pl.ds(t * L, L)]
            dst_v = didx[pl.ds(t * L, L)]
            et_v = etring[pl.ds(p * CHUNK + g * L, L)]
            cidx = jnp.right_shift(src_v, 9) * L + lane
            pos = plsc.load_gather(cursors, [cidx])
            plsc.store_scatter(cursors, [cidx], pos + 1)
            plsc.store_scatter(sidxp, [pos], src_v)
            plsc.store_scatter(didxp, [pos], dst_v)
            plsc.store_scatter(etp, [pos], et_v)
            plsc.store_scatter(permq, [pos], t * L + lane)
            return carry2

        lax.fori_loop(0, CHUNK // L, scat_vec, 0)
        return carry

    lax.fori_loop(0, NCHUNK, scat_chunk, 0)

    def w2r_body(t, carry):
        ko = t // L
        j = t - ko * L
        dl = jnp.bitwise_and(lane + j, L - 1)
        vals = plsc.load_gather(w2_v, [ko * L + dl])
        w2r[pl.ds(t * L, L)] = vals
        return carry

    lax.fori_loop(0, (D // L) * L, w2r_body, 0)

    def issue(ci, p):
        off = ci * CHUNK
        dsl = pl.ds(p * CHUNK, CHUNK)
        pltpu.async_copy(a_hbm.at[sidxp.at[pl.ds(off, CHUNK)]],
                         buf_a.at[dsl], gsem.at[p])
        pltpu.async_copy(b_hbm.at[didxp.at[pl.ds(off, CHUNK)]],
                         buf_b.at[dsl], gsem.at[p])
        pltpu.async_copy(c_hbm.at[etp.at[pl.ds(off, CHUNK)]],
                         buf_c.at[dsl], gsem.at[p])

    def drain(p):
        dsl = pl.ds(p * CHUNK, CHUNK)
        dummy = a_hbm.at[pl.ds(0, CHUNK)]
        pltpu.make_async_copy(dummy, buf_a.at[dsl], gsem.at[p]).wait()
        pltpu.make_async_copy(dummy, buf_b.at[dsl], gsem.at[p]).wait()
        pltpu.make_async_copy(dummy, buf_c.at[dsl], gsem.at[p]).wait()

    issue(0, 0)

    def chunk_body(ci, carry):
        p = lax.rem(ci, 2)

        @pl.when(ci + 1 < NCHUNK)
        def _():
            issue(ci + 1, 1 - p)

        drain(p)
        ebase = p * CHUNK

        def group_body(g, carry2):
            rows = ebase + g * L + lane

            def k_outer(ko, acc):
                fb = ko * L
                for j in range(L):
                    dl = jnp.bitwise_and(lane + j, L - 1)
                    fvec = fb + dl
                    va = plsc.load_gather(buf_a, [rows, fvec])
                    vb = plsc.load_gather(buf_b, [rows, fvec])
                    vc = plsc.load_gather(buf_c, [rows, fvec])
                    w2vec = w2r[pl.ds((fb + j) * L, L)]
                    acc = acc + jnp.maximum(va + vb + vc, 0.0) * w2vec
                return acc

            logv = lax.fori_loop(0, D // L, k_outer,
                                 jnp.full((L,), b2s, jnp.float32))
            origv = permq[pl.ds(ci * CHUNK + g * L, L)]
            plsc.store_scatter(lstage, [origv], logv)
            return carry2

        lax.fori_loop(0, CHUNK // L, group_body, 0)
        return carry

    lax.fori_loop(0, NCHUNK, chunk_body, 0)

    pltpu.sync_copy(lstage, logits_hbm.at[pl.ds(w_base, PER_W)])

    def sig_chunk(ci, carry):
        p = lax.rem(ci, 2)

        @pl.when(ci >= 2)
        def _():
            pltpu.make_async_copy(outring.at[pl.ds(p * CHUNK, CHUNK)],
                                  scores_hbm.at[pl.ds(w_base, CHUNK)],
                                  osem.at[p]).wait()

        def sig_vec(g, carry2):
            v = lstage[pl.ds(ci * CHUNK + g * L, L)]
            outring[pl.ds(p * CHUNK + g * L, L)] = 1.0 / (1.0 + jnp.exp(-v))
            return carry2

        lax.fori_loop(0, CHUNK // L, sig_vec, 0)
        pltpu.async_copy(outring.at[pl.ds(p * CHUNK, CHUNK)],
                         scores_hbm.at[pl.ds(w_base + ci * CHUNK, CHUNK)],
                         osem.at[p])
        return carry

    lax.fori_loop(0, NCHUNK, sig_chunk, 0)
    for p in range(2):
        pltpu.make_async_copy(outring.at[pl.ds(p * CHUNK, CHUNK)],
                              scores_hbm.at[pl.ds(w_base, CHUNK)],
                              osem.at[p]).wait()


@functools.partial(
    pl.kernel,
    out_type=[
        jax.ShapeDtypeStruct((EP,), jnp.float32),
        jax.ShapeDtypeStruct((EP,), jnp.float32),
    ],
    mesh=plsc.VectorSubcoreMesh(core_axis_name="c", subcore_axis_name="s"),
    compiler_params=pltpu.CompilerParams(needs_layout_passes=False),
    scratch_types=[
        pltpu.VMEM((PER_W,), jnp.int32),
        pltpu.VMEM((PER_W,), jnp.int32),
        pltpu.VMEM((PER_W,), jnp.int32),
        pltpu.VMEM((PER_W,), jnp.int32),
        pltpu.VMEM((PER_W,), jnp.int32),
        pltpu.VMEM((PER_W,), jnp.int32),
        pltpu.VMEM((PER_W,), jnp.float32),
        pltpu.VMEM((KB * L,), jnp.int32),
        pltpu.VMEM((KB * L,), jnp.int32),
        pltpu.VMEM((2 * CHUNK, D), jnp.float32),
        pltpu.VMEM((2 * CHUNK, D), jnp.float32),
        pltpu.VMEM((2 * CHUNK, D), jnp.float32),
        pltpu.VMEM((2 * CHUNK,), jnp.int32),
        pltpu.VMEM((2 * CHUNK,), jnp.float32),
        pltpu.VMEM((D,), jnp.float32),
        pltpu.VMEM((L,), jnp.float32),
        pltpu.VMEM((D * L,), jnp.float32),
        pltpu.SemaphoreType.DMA((2,)),
        pltpu.SemaphoreType.DMA((2,)),
    ],
)
def _sc_edges(a, b, c, src, dst, et, w2, b2, logits, scores, *scratch):
    _sc_edge_kernel(a, b, c, src, dst, et, w2, b2, logits, scores, *scratch)


def kernel(x, edge_index, edge_type, emb, W1, b1, W2, b2):
    a, b, c = _precompute_tables(x, emb, W1, b1)

    pad = EP - E
    src = jnp.concatenate([edge_index[0], jnp.zeros((pad,), jnp.int32)])
    dst = jnp.concatenate([edge_index[1], jnp.zeros((pad,), jnp.int32)])
    et = jnp.concatenate([edge_type, jnp.zeros((pad,), jnp.int32)])

    w2 = W2.reshape(D)
    b2v = jnp.broadcast_to(b2.reshape(1), (L,)).astype(jnp.float32)

    logits_p, scores_p = _sc_edges(a, b, c, src, dst, et, w2, b2v)
    return logits_p[:E], scores_p[:E]


# speedup vs baseline: 4.2587x; 1.3015x over previous
import functools

import jax
import jax.numpy as jnp
from jax import lax
from jax.experimental import pallas as pl
from jax.experimental.pallas import tpu as pltpu
from jax.experimental.pallas import tpu_sc as plsc

N = 10000
E = 320000
D = 128
ET = 16

NC = 2
NS = 16
NW = NC * NS
L = 16

CHUNK = 80
NBUF = 3
PER_W = 10240
EP = PER_W * NW
NCHUNK = PER_W // CHUNK


def _ab_body(x_ref, wa_ref, wb_ref, a_ref, b_ref):
    xb = x_ref[...]
    a_ref[...] = jnp.dot(xb, wa_ref[...], preferred_element_type=jnp.float32)
    b_ref[...] = jnp.dot(xb, wb_ref[...], preferred_element_type=jnp.float32)


def _c_body(emb_ref, wc_ref, b1_ref, c_ref):
    c_ref[...] = (
        jnp.dot(emb_ref[...], wc_ref[...], preferred_element_type=jnp.float32)
        + b1_ref[...]
    )


def _precompute_tables(x, emb, W1, b1):
    w1a = W1[0:D, :]
    w1b = W1[D:2 * D, :]
    w1c = W1[2 * D:, :]
    blk = 1000
    a, b = pl.pallas_call(
        _ab_body,
        grid=(N // blk,),
        in_specs=[
            pl.BlockSpec((blk, D), lambda i: (i, 0)),
            pl.BlockSpec((D, D), lambda i: (0, 0)),
            pl.BlockSpec((D, D), lambda i: (0, 0)),
        ],
        out_specs=[
            pl.BlockSpec((blk, D), lambda i: (i, 0)),
            pl.BlockSpec((blk, D), lambda i: (i, 0)),
        ],
        out_shape=[
            jax.ShapeDtypeStruct((N, D), jnp.float32),
            jax.ShapeDtypeStruct((N, D), jnp.float32),
        ],
    )(x, w1a, w1b)
    c = pl.pallas_call(
        _c_body,
        out_shape=jax.ShapeDtypeStruct((512, D), jnp.float32),
    )(emb, w1c, b1.reshape(1, D))
    return a, b, c


def _sc_edge_kernel(a_hbm, b_hbm, c_hbm, src_hbm, dst_hbm, et_hbm,
                    w2_hbm, b2_hbm, logits_hbm, scores_hbm,
                    sidx, didx, etr, buf_a, buf_b, c_v,
                    lring, sring, w2_v, b2_v, w2r, gsem, osem):
    wid = lax.axis_index("s") * NC + lax.axis_index("c")
    w_base = wid * PER_W

    pltpu.sync_copy(w2_hbm, w2_v)
    pltpu.sync_copy(b2_hbm, b2_v)
    pltpu.sync_copy(c_hbm, c_v)
    pltpu.sync_copy(src_hbm.at[pl.ds(w_base, PER_W)], sidx)
    pltpu.sync_copy(dst_hbm.at[pl.ds(w_base, PER_W)], didx)

    b2s = b2_v[...][0]
    lane = lax.broadcasted_iota(jnp.int32, (L,), 0)

    def w2r_body(t, carry):
        ko = t // L
        j = t - ko * L
        dl = jnp.bitwise_and(lane + j, L - 1)
        vals = plsc.load_gather(w2_v, [ko * L + dl])
        w2r[pl.ds(t * L, L)] = vals
        return carry

    lax.fori_loop(0, (D // L) * L, w2r_body, 0)

    H = CHUNK // 2

    def issue(ci, p):
        off = ci * CHUNK
        pltpu.async_copy(a_hbm.at[sidx.at[pl.ds(off, H)]],
                         buf_a.at[pl.ds(p * CHUNK, H)], gsem.at[p])
        pltpu.async_copy(a_hbm.at[sidx.at[pl.ds(off + H, H)]],
                         buf_a.at[pl.ds(p * CHUNK + H, H)], gsem.at[p])
        pltpu.async_copy(b_hbm.at[didx.at[pl.ds(off, H)]],
                         buf_b.at[pl.ds(p * CHUNK, H)], gsem.at[p])
        pltpu.async_copy(b_hbm.at[didx.at[pl.ds(off + H, H)]],
                         buf_b.at[pl.ds(p * CHUNK + H, H)], gsem.at[p])
        pltpu.async_copy(et_hbm.at[pl.ds(w_base + off, CHUNK)],
                         etr.at[pl.ds(p * CHUNK, CHUNK)], gsem.at[p])

    def drain(p):
        dummy = a_hbm.at[pl.ds(0, H)]
        pltpu.make_async_copy(dummy, buf_a.at[pl.ds(p * CHUNK, H)],
                              gsem.at[p]).wait()
        pltpu.make_async_copy(dummy, buf_a.at[pl.ds(p * CHUNK + H, H)],
                              gsem.at[p]).wait()
        pltpu.make_async_copy(dummy, buf_b.at[pl.ds(p * CHUNK, H)],
                              gsem.at[p]).wait()
        pltpu.make_async_copy(dummy, buf_b.at[pl.ds(p * CHUNK + H, H)],
                              gsem.at[p]).wait()
        pltpu.make_async_copy(et_hbm.at[pl.ds(0, CHUNK)],
                              etr.at[pl.ds(p * CHUNK, CHUNK)],
                              gsem.at[p]).wait()

    def out_issue(ci, p):
        base = w_base + ci * CHUNK
        dsl = pl.ds(p * CHUNK, CHUNK)
        pltpu.async_copy(lring.at[dsl], logits_hbm.at[pl.ds(base, CHUNK)],
                         osem.at[p])
        pltpu.async_copy(sring.at[dsl], scores_hbm.at[pl.ds(base, CHUNK)],
                         osem.at[p])

    def out_drain(p):
        dsl = pl.ds(p * CHUNK, CHUNK)
        pltpu.make_async_copy(lring.at[dsl],
                              logits_hbm.at[pl.ds(w_base, CHUNK)],
                              osem.at[p]).wait()
        pltpu.make_async_copy(sring.at[dsl],
                              scores_hbm.at[pl.ds(w_base, CHUNK)],
                              osem.at[p]).wait()

    for w in range(NBUF - 1):
        issue(w, w)

    def chunk_body(ci, carry):
        p = lax.rem(ci, NBUF)

        @pl.when(ci + NBUF - 1 < NCHUNK)
        def _():
            issue(ci + NBUF - 1, lax.rem(ci + NBUF - 1, NBUF))

        drain(p)

        @pl.when(ci >= NBUF)
        def _():
            out_drain(p)

        ebase = p * CHUNK

        def group_body(g, carry2):
            rows = ebase + g * L + lane
            etv = etr[pl.ds(ebase + g * L, L)]

            def k_outer(ko, acc):
                fb = ko * L
                for j in range(L):
                    dl = jnp.bitwise_and(lane + j, L - 1)
                    fvec = fb + dl
                    va = plsc.load_gather(buf_a, [rows, fvec])
                    vb = plsc.load_gather(buf_b, [rows, fvec])
                    wcol = (fb // 2) + jnp.right_shift(dl, 1)
                    cw = plsc.load_gather(c_v, [etv * (D // 2) + wcol])
                    clo, chi = plsc.unpack(
                        plsc.bitcast(cw, jnp.bfloat16),
                        format=plsc.PackFormat.INTERLEAVED)
                    odd = jnp.bitwise_and(lane + j, 1) == 1
                    vc = jnp.where(odd, chi, clo)
                    w2vec = w2r[pl.ds((fb + j) * L, L)]
                    acc = acc + jnp.maximum(va + vb + vc, 0.0) * w2vec
                return acc

            logv = lax.fori_loop(0, D // L, k_outer,
                                 jnp.full((L,), b2s, jnp.float32))
            obase = ebase + g * L
            lring[pl.ds(obase, L)] = logv
            sring[pl.ds(obase, L)] = 1.0 / (1.0 + jnp.exp(-logv))
            return carry2

        lax.fori_loop(0, CHUNK // L, group_body, 0)
        out_issue(ci, p)
        return carry

    lax.fori_loop(0, NCHUNK, chunk_body, 0)
    for w in range(NBUF):
        out_drain((NCHUNK - NBUF + w) % NBUF)


@functools.partial(
    pl.kernel,
    out_type=[
        jax.ShapeDtypeStruct((EP,), jnp.float32),
        jax.ShapeDtypeStruct((EP,), jnp.float32),
    ],
    mesh=plsc.VectorSubcoreMesh(core_axis_name="c", subcore_axis_name="s"),
    compiler_params=pltpu.CompilerParams(needs_layout_passes=False),
    scratch_types=[
        pltpu.VMEM((PER_W,), jnp.int32),
        pltpu.VMEM((PER_W,), jnp.int32),
        pltpu.VMEM((NBUF * CHUNK,), jnp.int32),
        pltpu.VMEM((NBUF * CHUNK, D), jnp.float32),
        pltpu.VMEM((NBUF * CHUNK, D), jnp.float32),
        pltpu.VMEM((512 * D // 2,), jnp.int32),
        pltpu.VMEM((NBUF * CHUNK,), jnp.float32),
        pltpu.VMEM((NBUF * CHUNK,), jnp.float32),
        pltpu.VMEM((D,), jnp.float32),
        pltpu.VMEM((L,), jnp.float32),
        pltpu.VMEM((D * L,), jnp.float32),
        pltpu.SemaphoreType.DMA((NBUF,)),
        pltpu.SemaphoreType.DMA((NBUF,)),
    ],
)
def _sc_edges(a, b, c, src, dst, et, w2, b2, logits, scores, *scratch):
    _sc_edge_kernel(a, b, c, src, dst, et, w2, b2, logits, scores, *scratch)


def kernel(x, edge_index, edge_type, emb, W1, b1, W2, b2):
    a, b, c = _precompute_tables(x, emb, W1, b1)
    c = lax.bitcast_convert_type(
        c.astype(jnp.bfloat16).reshape(512, D // 2, 2), jnp.int32).reshape(-1)

    pad = EP - E
    src = jnp.concatenate([edge_index[0], jnp.zeros((pad,), jnp.int32)])
    dst = jnp.concatenate([edge_index[1], jnp.zeros((pad,), jnp.int32)])
    et = jnp.concatenate([edge_type, jnp.zeros((pad,), jnp.int32)])

    w2 = W2.reshape(D)
    b2v = jnp.broadcast_to(b2.reshape(1), (L,)).astype(jnp.float32)

    logits_p, scores_p = _sc_edges(a, b, c, src, dst, et, w2, b2v)
    return logits_p[:E], scores_p[:E]


# speedup vs baseline: 4.2881x; 1.0069x over previous
import functools

import jax
import jax.numpy as jnp
from jax import lax
from jax.experimental import pallas as pl
from jax.experimental.pallas import tpu as pltpu
from jax.experimental.pallas import tpu_sc as plsc

N = 10000
E = 320000
D = 128
ET = 16

NC = 2
NS = 16
NW = NC * NS
L = 16

CHUNK = 128
NBUF = 2
PER_W = 10240
EP = PER_W * NW
NCHUNK = PER_W // CHUNK


def _ab_body(x_ref, wa_ref, wb_ref, a_ref, b_ref):
    xb = x_ref[...]
    a_ref[...] = jnp.dot(xb, wa_ref[...], preferred_element_type=jnp.float32)
    b_ref[...] = jnp.dot(xb, wb_ref[...], preferred_element_type=jnp.float32)


def _c_body(emb_ref, wc_ref, b1_ref, c_ref):
    c_ref[...] = (
        jnp.dot(emb_ref[...], wc_ref[...], preferred_element_type=jnp.float32)
        + b1_ref[...]
    )


def _precompute_tables(x, emb, W1, b1):
    w1a = W1[0:D, :]
    w1b = W1[D:2 * D, :]
    w1c = W1[2 * D:, :]
    blk = 1000
    a, b = pl.pallas_call(
        _ab_body,
        grid=(N // blk,),
        in_specs=[
            pl.BlockSpec((blk, D), lambda i: (i, 0)),
            pl.BlockSpec((D, D), lambda i: (0, 0)),
            pl.BlockSpec((D, D), lambda i: (0, 0)),
        ],
        out_specs=[
            pl.BlockSpec((blk, D), lambda i: (i, 0)),
            pl.BlockSpec((blk, D), lambda i: (i, 0)),
        ],
        out_shape=[
            jax.ShapeDtypeStruct((N, D), jnp.float32),
            jax.ShapeDtypeStruct((N, D), jnp.float32),
        ],
    )(x, w1a, w1b)
    c = pl.pallas_call(
        _c_body,
        out_shape=jax.ShapeDtypeStruct((512, D), jnp.float32),
    )(emb, w1c, b1.reshape(1, D))
    return a, b, c


def _sc_edge_kernel(a_hbm, b_hbm, c_hbm, src_hbm, dst_hbm, et_hbm,
                    w2_hbm, b2_hbm, logits_hbm, scores_hbm,
                    sidx, didx, etr, buf_a, buf_b, c_v,
                    lring, sring, w2_v, b2_v, w2r, gsem, osem):
    wid = lax.axis_index("s") * NC + lax.axis_index("c")
    w_base = wid * PER_W

    pltpu.sync_copy(w2_hbm, w2_v)
    pltpu.sync_copy(b2_hbm, b2_v)
    pltpu.sync_copy(c_hbm, c_v)
    pltpu.sync_copy(src_hbm.at[pl.ds(w_base, PER_W)], sidx)
    pltpu.sync_copy(dst_hbm.at[pl.ds(w_base, PER_W)], didx)

    b2s = b2_v[...][0]
    lane = lax.broadcasted_iota(jnp.int32, (L,), 0)

    def w2r_body(t, carry):
        ko = t // L
        j = t - ko * L
        dl = jnp.bitwise_and(lane + j, L - 1)
        vals = plsc.load_gather(w2_v, [ko * L + dl])
        w2r[pl.ds(t * L, L)] = vals
        return carry

    lax.fori_loop(0, (D // L) * L, w2r_body, 0)

    H = CHUNK // 2

    def issue(ci, p):
        off = ci * CHUNK
        pltpu.async_copy(a_hbm.at[sidx.at[pl.ds(off, H)]],
                         buf_a.at[pl.ds(p * CHUNK, H)], gsem.at[p])
        pltpu.async_copy(a_hbm.at[sidx.at[pl.ds(off + H, H)]],
                         buf_a.at[pl.ds(p * CHUNK + H, H)], gsem.at[p])
        pltpu.async_copy(b_hbm.at[didx.at[pl.ds(off, H)]],
                         buf_b.at[pl.ds(p * CHUNK, H)], gsem.at[p])
        pltpu.async_copy(b_hbm.at[didx.at[pl.ds(off + H, H)]],
                         buf_b.at[pl.ds(p * CHUNK + H, H)], gsem.at[p])
        pltpu.async_copy(et_hbm.at[pl.ds(w_base + off, CHUNK)],
                         etr.at[pl.ds(p * CHUNK, CHUNK)], gsem.at[p])

    def drain(p):
        dummy = a_hbm.at[pl.ds(0, H)]
        pltpu.make_async_copy(dummy, buf_a.at[pl.ds(p * CHUNK, H)],
                              gsem.at[p]).wait()
        pltpu.make_async_copy(dummy, buf_a.at[pl.ds(p * CHUNK + H, H)],
                              gsem.at[p]).wait()
        pltpu.make_async_copy(dummy, buf_b.at[pl.ds(p * CHUNK, H)],
                              gsem.at[p]).wait()
        pltpu.make_async_copy(dummy, buf_b.at[pl.ds(p * CHUNK + H, H)],
                              gsem.at[p]).wait()
        pltpu.make_async_copy(et_hbm.at[pl.ds(0, CHUNK)],
                              etr.at[pl.ds(p * CHUNK, CHUNK)],
                              gsem.at[p]).wait()

    def out_issue(ci, p):
        base = w_base + ci * CHUNK
        dsl = pl.ds(p * CHUNK, CHUNK)
        pltpu.async_copy(lring.at[dsl], logits_hbm.at[pl.ds(base, CHUNK)],
                         osem.at[p])
        pltpu.async_copy(sring.at[dsl], scores_hbm.at[pl.ds(base, CHUNK)],
                         osem.at[p])

    def out_drain(p):
        dsl = pl.ds(p * CHUNK, CHUNK)
        pltpu.make_async_copy(lring.at[dsl],
                              logits_hbm.at[pl.ds(w_base, CHUNK)],
                              osem.at[p]).wait()
        pltpu.make_async_copy(sring.at[dsl],
                              scores_hbm.at[pl.ds(w_base, CHUNK)],
                              osem.at[p]).wait()

    for w in range(NBUF - 1):
        issue(w, w)

    def chunk_body(ci, carry):
        p = lax.rem(ci, NBUF)

        @pl.when(ci + NBUF - 1 < NCHUNK)
        def _():
            issue(ci + NBUF - 1, lax.rem(ci + NBUF - 1, NBUF))

        drain(p)

        @pl.when(ci >= NBUF)
        def _():
            out_drain(p)

        ebase = p * CHUNK

        def group_body(g, carry2):
            rows = ebase + g * L + lane
            etv = etr[pl.ds(ebase + g * L, L)]

            def k_outer(ko, acc):
                fb = ko * L
                for j in range(L):
                    dl = jnp.bitwise_and(lane + j, L - 1)
                    fvec = fb + dl
                    va = plsc.load_gather(buf_a, [rows, fvec])
                    vb = plsc.load_gather(buf_b, [rows, fvec])
                    wcol = (fb // 2) + jnp.right_shift(dl, 1)
                    cw = plsc.load_gather(c_v, [etv * (D // 2) + wcol])
                    clo, chi = plsc.unpack(
                        plsc.bitcast(cw, jnp.bfloat16),
                        format=plsc.PackFormat.INTERLEAVED)
                    odd = jnp.bitwise_and(lane + j, 1) == 1
                    vc = jnp.where(odd, chi, clo)
                    w2vec = w2r[pl.ds((fb + j) * L, L)]
                    acc = acc + jnp.maximum(va + vb + vc, 0.0) * w2vec
                return acc

            logv = lax.fori_loop(0, D // L, k_outer,
                                 jnp.full((L,), b2s, jnp.float32))
            obase = ebase + g * L
            lring[pl.ds(obase, L)] = logv
            sring[pl.ds(obase, L)] = 1.0 / (1.0 + jnp.exp(-logv))
            return carry2

        lax.fori_loop(0, CHUNK // L, group_body, 0)
        out_issue(ci, p)
        return carry

    lax.fori_loop(0, NCHUNK, chunk_body, 0)
    for w in range(NBUF):
        out_drain((NCHUNK - NBUF + w) % NBUF)


@functools.partial(
    pl.kernel,
    out_type=[
        jax.ShapeDtypeStruct((EP,), jnp.float32),
        jax.ShapeDtypeStruct((EP,), jnp.float32),
    ],
    mesh=plsc.VectorSubcoreMesh(core_axis_name="c", subcore_axis_name="s"),
    compiler_params=pltpu.CompilerParams(needs_layout_passes=False),
    scratch_types=[
        pltpu.VMEM((PER_W,), jnp.int32),
        pltpu.VMEM((PER_W,), jnp.int32),
        pltpu.VMEM((NBUF * CHUNK,), jnp.int32),
        pltpu.VMEM((NBUF * CHUNK, D), jnp.float32),
        pltpu.VMEM((NBUF * CHUNK, D), jnp.float32),
        pltpu.VMEM((512 * D // 2,), jnp.int32),
        pltpu.VMEM((NBUF * CHUNK,), jnp.float32),
        pltpu.VMEM((NBUF * CHUNK,), jnp.float32),
        pltpu.VMEM((D,), jnp.float32),
        pltpu.VMEM((L,), jnp.float32),
        pltpu.VMEM((D * L,), jnp.float32),
        pltpu.SemaphoreType.DMA((NBUF,)),
        pltpu.SemaphoreType.DMA((NBUF,)),
    ],
)
def _sc_edges(a, b, c, src, dst, et, w2, b2, logits, scores, *scratch):
    _sc_edge_kernel(a, b, c, src, dst, et, w2, b2, logits, scores, *scratch)


def kernel(x, edge_index, edge_type, emb, W1, b1, W2, b2):
    a, b, c = _precompute_tables(x, emb, W1, b1)
    c = lax.bitcast_convert_type(
        c.astype(jnp.bfloat16).reshape(512, D // 2, 2), jnp.int32).reshape(-1)

    pad = EP - E
    src = jnp.concatenate([edge_index[0], jnp.zeros((pad,), jnp.int32)])
    dst = jnp.concatenate([edge_index[1], jnp.zeros((pad,), jnp.int32)])
    et = jnp.concatenate([edge_type, jnp.zeros((pad,), jnp.int32)])

    w2 = W2.reshape(D)
    b2v = jnp.broadcast_to(b2.reshape(1), (L,)).astype(jnp.float32)

    logits_p, scores_p = _sc_edges(a, b, c, src, dst, et, w2, b2v)
    return logits_p[:E], scores_p[:E]
